# baseline (speedup 1.0000x reference)
import functools

import jax
import jax.numpy as jnp
import numpy as np
from jax import lax
from jax.experimental import pallas as pl
from jax.experimental.pallas import tpu as pltpu
from jax.experimental.pallas import tpu_sc as plsc

BS_, NQ_, D_ = 4, 5440, 256
H_, L_, P_ = 8, 4, 4
HD_ = D_ // H_
SH_ = np.array([[64, 64], [32, 32], [16, 16], [8, 8]], dtype=np.int64)
NV_ = int((SH_[:, 0] * SH_[:, 1]).sum())
STARTS_ = np.concatenate([[0], np.cumsum(SH_[:, 0] * SH_[:, 1])[:-1]]).astype(np.int64)
NG_ = BS_ * NQ_
TQ_ = 1360
NBLK_ = NG_ // TQ_
NW_ = 32
GPW_ = NG_ // NW_

_lane = np.arange(H_ * L_ * P_)
_l_of = (_lane // P_) % L_
_W_I = SH_[_l_of, 1].astype(np.int32)
_H_I = SH_[_l_of, 0].astype(np.int32)
_START_I = STARTS_[_l_of].astype(np.int32)
_HEAD_I = (_lane // (L_ * P_)).astype(np.int32)
_BGRP = (np.arange(128)[:, None] // (L_ * P_) == np.arange(128)[None, :] // (L_ * P_)).astype(np.float32)


def _matmul_body(x_ref, w_ref, b_ref, o_ref):
  r = jnp.dot(x_ref[...], w_ref[...], preferred_element_type=jnp.float32) + b_ref[0]
  o_ref[...] = r.astype(o_ref.dtype)


def _proj(x, w, b, out_dtype=jnp.float32):
  n = x.shape[0]
  return pl.pallas_call(
      _matmul_body,
      grid=(n // TQ_,),
      in_specs=[
          pl.BlockSpec((TQ_, x.shape[1]), lambda g: (g, 0)),
          pl.BlockSpec(w.shape, lambda g: (0, 0)),
          pl.BlockSpec((1, b.shape[1]), lambda g: (0, 0)),
      ],
      out_specs=pl.BlockSpec((TQ_, w.shape[1]), lambda g: (g, 0)),
      out_shape=jax.ShapeDtypeStruct((n, w.shape[1]), out_dtype),
  )(x, w, b)


def _residual_body(x_ref, w_ref, b_ref, q_ref, o_ref):
  o_ref[...] = (jnp.dot(x_ref[...], w_ref[...], preferred_element_type=jnp.float32)
                + b_ref[0] + q_ref[...])


def _out_proj(x, w, b, q):
  n = x.shape[0]
  return pl.pallas_call(
      _residual_body,
      grid=(n // TQ_,),
      in_specs=[
          pl.BlockSpec((TQ_, D_), lambda g: (g, 0)),
          pl.BlockSpec((D_, D_), lambda g: (0, 0)),
          pl.BlockSpec((1, D_), lambda g: (0, 0)),
          pl.BlockSpec((TQ_, D_), lambda g: (g, 0)),
      ],
      out_specs=pl.BlockSpec((TQ_, D_), lambda g: (g, 0)),
      out_shape=jax.ShapeDtypeStruct((n, D_), jnp.float32),
  )(x, w, b, q)


def _sampling_body(b0, q_ref, rpx_ref, rpy_ref, wox_ref, woy_ref, wat_ref,
                   box_ref, boy_ref, bat_ref, bgrp_ref, lc_ref, iw_ref):
  q = q_ref[...]
  offx = jnp.dot(q, wox_ref[...], preferred_element_type=jnp.float32) + box_ref[0]
  offy = jnp.dot(q, woy_ref[...], preferred_element_type=jnp.float32) + boy_ref[0]
  a = jnp.dot(q, wat_ref[...], preferred_element_type=jnp.float32) + bat_ref[0]
  m = jnp.max(a, axis=-1, keepdims=True)
  e = jnp.exp(a - m)
  s = jnp.dot(e, bgrp_ref[...], preferred_element_type=jnp.float32)
  aw = e / s

  x = rpx_ref[...] + offx
  y = rpy_ref[...] + offy
  x0f = jnp.floor(x)
  y0f = jnp.floor(y)
  fx = x - x0f
  fy = y - y0f
  x0 = x0f.astype(jnp.int32)
  y0 = y0f.astype(jnp.int32)

  wl = lc_ref[0:1, :]
  hl = lc_ref[1:2, :]
  st = lc_ref[2:3, :]
  hh = lc_ref[3:4, :]
  b = b0 + pl.program_id(0) // (NQ_ // TQ_)
  base = (b * (NV_ * H_)).astype(jnp.int32)

  corners = (
      (0, 0, (1.0 - fx) * (1.0 - fy)),
      (1, 0, fx * (1.0 - fy)),
      (0, 1, (1.0 - fx) * fy),
      (1, 1, fx * fy),
  )
  for c, (dx, dy, wgt) in enumerate(corners):
    xi = x0 + dx
    yi = y0 + dy
    valid = ((xi >= 0) & (xi <= wl - 1) & (yi >= 0) & (yi <= hl - 1))
    xc = jnp.clip(xi, 0, wl - 1)
    yc = jnp.clip(yi, 0, hl - 1)
    iw_ref[c] = (st + yc * wl + xc) * H_ + hh + base
    wu = lax.bitcast_convert_type(
        (aw * wgt * valid.astype(jnp.float32)).astype(jnp.bfloat16),
        jnp.uint16).astype(jnp.uint32)
    iw_ref[4 + c] = lax.bitcast_convert_type((wu << 16) | wu, jnp.int32)


def _sampling(q2, rpx, rpy, wox, woy, wat, box, boy, bat, b0=0):
  ng = q2.shape[0]
  return pl.pallas_call(
      functools.partial(_sampling_body, b0),
      grid=(ng // TQ_,),
      in_specs=[
          pl.BlockSpec((TQ_, D_), lambda g: (g, 0)),
          pl.BlockSpec((TQ_, 128), lambda g: (g, 0)),
          pl.BlockSpec((TQ_, 128), lambda g: (g, 0)),
          pl.BlockSpec((D_, 128), lambda g: (0, 0)),
          pl.BlockSpec((D_, 128), lambda g: (0, 0)),
          pl.BlockSpec((D_, 128), lambda g: (0, 0)),
          pl.BlockSpec((1, 128), lambda g: (0, 0)),
          pl.BlockSpec((1, 128), lambda g: (0, 0)),
          pl.BlockSpec((1, 128), lambda g: (0, 0)),
          pl.BlockSpec((128, 128), lambda g: (0, 0)),
          pl.BlockSpec((4, 128), lambda g: (0, 0)),
      ],
      out_specs=pl.BlockSpec((8, TQ_, 128), lambda g: (0, g, 0)),
      out_shape=jax.ShapeDtypeStruct((8, ng, 128), jnp.int32),
  )(q2, rpx, rpy, wox, woy, wat, box, boy, bat, jnp.asarray(_BGRP),
    jnp.asarray(np.stack([_W_I, _H_I, _START_I, _HEAD_I])))


_NS = 4


def _sc_gather(table, iw):
  mesh = plsc.VectorSubcoreMesh(core_axis_name="c", subcore_axis_name="s")
  ng = iw.shape[1]
  gpw = ng // NW_
  nit = gpw // _NS

  @functools.partial(
      pl.kernel,
      out_type=jax.ShapeDtypeStruct((ng, D_), jnp.float32),
      mesh=mesh,
      scratch_types=[
          [pltpu.VMEM((8, 128), jnp.int32)] * _NS,
          [[pltpu.VMEM((128, HD_), jnp.bfloat16)] * 4] * _NS,
          [pltpu.VMEM((D_,), jnp.float32)] * _NS,
          [pltpu.SemaphoreType.DMA] * _NS,
          [pltpu.SemaphoreType.DMA] * _NS,
          [pltpu.SemaphoreType.DMA] * _NS,
      ],
      compiler_params=pltpu.CompilerParams(use_tc_tiling_on_sc=False,
                                           needs_layout_passes=False),
  )
  def k(table_hbm, iw_hbm, out_hbm, iws, rows, os, sg, st, so):
    wid = lax.axis_index("s") * 2 + lax.axis_index("c")
    base = wid * gpw

    def stage(g, s):
      pltpu.async_copy(iw_hbm.at[:, g], iws[s], st[s])

    def wait_stage(g, s):
      pltpu.make_async_copy(iw_hbm.at[:, g], iws[s], st[s]).wait()

    def gathers(s):
      for c in range(4):
        pltpu.async_copy(table_hbm.at[iws[s].at[c]], rows[s][c], sg[s])

    def wait_gathers(s):
      for c in range(4):
        pltpu.make_async_copy(table_hbm.at[iws[s].at[c]], rows[s][c], sg[s]).wait()

    def compute(s):
      w_v = iws[s]
      out_v = os[s]
      for h in range(H_):
        acc = []
        for c in range(4):
          wv = w_v[4 + c, pl.ds(h * 16, 16)]
          rr = rows[s][c]
          pc = [jnp.zeros((32,), jnp.bfloat16), jnp.zeros((32,), jnp.bfloat16)]
          for j in range(16):
            wj = lax.gather(
                wv, jnp.full((16, 1), j, jnp.int32),
                lax.GatherDimensionNumbers(offset_dims=(), collapsed_slice_dims=(0,),
                                           start_index_map=(0,)),
                (1,), mode=lax.GatherScatterMode.PROMISE_IN_BOUNDS)
            wj32 = plsc.bitcast(wj, jnp.bfloat16)
            pc[j % 2] = pc[j % 2] + wj32 * rr[h * 16 + j, :]
          acc.append(pc[0] + pc[1])
        e0, o0 = plsc.unpack(acc[0], format=plsc.PackFormat.INTERLEAVED)
        e1, o1 = plsc.unpack(acc[1], format=plsc.PackFormat.INTERLEAVED)
        e2, o2 = plsc.unpack(acc[2], format=plsc.PackFormat.INTERLEAVED)
        e3, o3 = plsc.unpack(acc[3], format=plsc.PackFormat.INTERLEAVED)
        out_v[pl.ds(h * HD_, 16)] = (e0 + e1) + (e2 + e3)
        out_v[pl.ds(h * HD_ + 16, 16)] = (o0 + o1) + (o2 + o3)

    for s in range(_NS):
      stage(base + s, s)
    wait_stage(base, 0)
    gathers(0)

    def body(i, carry):
      g0 = base + _NS * i
      for s in range(_NS):
        g = g0 + s
        sn = (s + 1) % _NS
        if s < _NS - 1:
          wait_stage(g + 1, sn)
          gathers(sn)
        else:
          @pl.when(i < nit - 1)
          def _():
            wait_stage(g + 1, sn)
            gathers(sn)
        wait_gathers(s)

        @pl.when(i > 0)
        def _():
          pltpu.make_async_copy(os[s], out_hbm.at[g - _NS], so[s]).wait()
        compute(s)
        pltpu.async_copy(os[s], out_hbm.at[g], so[s])

        @pl.when(i < nit - 1)
        def _():
          stage(g + _NS, s)
      return carry

    lax.fori_loop(0, nit, body, 0)
    for s in range(_NS):
      pltpu.make_async_copy(os[s], out_hbm.at[base + gpw - _NS + s], so[s]).wait()

  return k(table, iw)


def kernel(query, value, reference_points, spatial_shapes, level_start_index,
           W_value, b_value, W_offsets, b_offsets, W_attn, b_attn, W_out, b_out):
  cperm = (np.arange(D_).reshape(H_, 2, 16).transpose(0, 2, 1).reshape(-1))
  wv_sw = W_value[:, cperm]
  bv_sw = b_value[cperm].reshape(1, D_)
  wox = W_offsets[:, 0::2]
  woy = W_offsets[:, 1::2]
  box = b_offsets[0::2].reshape(1, 128)
  boy = b_offsets[1::2].reshape(1, 128)
  bat = b_attn.reshape(1, 128)
  bo2 = b_out.reshape(1, D_)
  lane_l = jnp.asarray(_l_of.astype(np.int32))
  wl_f = jnp.asarray(_W_I.astype(np.float32))[None, None, :]
  hl_f = jnp.asarray(_H_I.astype(np.float32))[None, None, :]

  outs = []
  for b0 in (0, 2):
    qh = query[b0:b0 + 2].reshape(2 * NQ_, D_)
    vh = value[b0:b0 + 2].reshape(2 * NV_, D_)
    rph = reference_points[b0:b0 + 2]
    table = _proj(vh, wv_sw, bv_sw,
                  out_dtype=jnp.bfloat16).reshape(2 * NV_ * H_, HD_)
    rpx = (jnp.take(rph[..., 0], lane_l, axis=2) * wl_f - 0.5).reshape(2 * NQ_, 128)
    rpy = (jnp.take(rph[..., 1], lane_l, axis=2) * hl_f - 0.5).reshape(2 * NQ_, 128)
    iw = _sampling(qh, rpx, rpy, wox, woy, W_attn, box, boy, bat)
    sampled = _sc_gather(table, iw)
    outs.append(_out_proj(sampled, W_out, bo2, qh))
  return jnp.concatenate(outs, axis=0).reshape(BS_, NQ_, D_)


# speedup vs baseline: 1.1130x; 1.1130x over previous
import functools

import jax
import jax.numpy as jnp
import numpy as np
from jax import lax
from jax.experimental import pallas as pl
from jax.experimental.pallas import tpu as pltpu
from jax.experimental.pallas import tpu_sc as plsc

BS_, NQ_, D_ = 4, 5440, 256
H_, L_, P_ = 8, 4, 4
HD_ = D_ // H_
SH_ = np.array([[64, 64], [32, 32], [16, 16], [8, 8]], dtype=np.int64)
NV_ = int((SH_[:, 0] * SH_[:, 1]).sum())
STARTS_ = np.concatenate([[0], np.cumsum(SH_[:, 0] * SH_[:, 1])[:-1]]).astype(np.int64)
NG_ = BS_ * NQ_
TQ_ = 1360
NBLK_ = NG_ // TQ_
NW_ = 32
GPW_ = NG_ // NW_

_lane = np.arange(H_ * L_ * P_)
_l_of = (_lane // P_) % L_
_W_I = SH_[_l_of, 1].astype(np.int32)
_H_I = SH_[_l_of, 0].astype(np.int32)
_START_I = STARTS_[_l_of].astype(np.int32)
_HEAD_I = (_lane // (L_ * P_)).astype(np.int32)
_BGRP = (np.arange(128)[:, None] // (L_ * P_) == np.arange(128)[None, :] // (L_ * P_)).astype(np.float32)


def _matmul_body(x_ref, w_ref, b_ref, o_ref):
  r = jnp.dot(x_ref[...], w_ref[...], preferred_element_type=jnp.float32) + b_ref[0]
  o_ref[...] = r.astype(o_ref.dtype)


def _proj(x, w, b, out_dtype=jnp.float32):
  n = x.shape[0]
  return pl.pallas_call(
      _matmul_body,
      grid=(n // TQ_,),
      in_specs=[
          pl.BlockSpec((TQ_, x.shape[1]), lambda g: (g, 0)),
          pl.BlockSpec(w.shape, lambda g: (0, 0)),
          pl.BlockSpec((1, b.shape[1]), lambda g: (0, 0)),
      ],
      out_specs=pl.BlockSpec((TQ_, w.shape[1]), lambda g: (g, 0)),
      out_shape=jax.ShapeDtypeStruct((n, w.shape[1]), out_dtype),
  )(x, w, b)


def _residual_body(x_ref, w_ref, b_ref, q_ref, o_ref):
  o_ref[...] = (jnp.dot(x_ref[...], w_ref[...], preferred_element_type=jnp.float32)
                + b_ref[0] + q_ref[...])


def _out_proj(x, w, b, q):
  n = x.shape[0]
  return pl.pallas_call(
      _residual_body,
      grid=(n // TQ_,),
      in_specs=[
          pl.BlockSpec((TQ_, D_), lambda g: (g, 0)),
          pl.BlockSpec((D_, D_), lambda g: (0, 0)),
          pl.BlockSpec((1, D_), lambda g: (0, 0)),
          pl.BlockSpec((TQ_, D_), lambda g: (g, 0)),
      ],
      out_specs=pl.BlockSpec((TQ_, D_), lambda g: (g, 0)),
      out_shape=jax.ShapeDtypeStruct((n, D_), jnp.float32),
  )(x, w, b, q)


def _sampling_body(q_ref, rpx_ref, rpy_ref, wox_ref, woy_ref, wat_ref,
                   box_ref, boy_ref, bat_ref, bgrp_ref, lc_ref, iw_ref):
  q = q_ref[...]
  offx = jnp.dot(q, wox_ref[...], preferred_element_type=jnp.float32) + box_ref[0]
  offy = jnp.dot(q, woy_ref[...], preferred_element_type=jnp.float32) + boy_ref[0]
  a = jnp.dot(q, wat_ref[...], preferred_element_type=jnp.float32) + bat_ref[0]
  m = jnp.max(a, axis=-1, keepdims=True)
  e = jnp.exp(a - m)
  s = jnp.dot(e, bgrp_ref[...], preferred_element_type=jnp.float32)
  aw = e / s

  x = rpx_ref[...] + offx
  y = rpy_ref[...] + offy
  x0f = jnp.floor(x)
  y0f = jnp.floor(y)
  fx = x - x0f
  fy = y - y0f
  x0 = x0f.astype(jnp.int32)
  y0 = y0f.astype(jnp.int32)

  wl = lc_ref[0:1, :]
  hl = lc_ref[1:2, :]
  st = lc_ref[2:3, :]
  hh = lc_ref[3:4, :]
  b = pl.program_id(0) // (NQ_ // TQ_)
  base = (b * (NV_ * H_)).astype(jnp.int32)

  corners = (
      (0, 0, (1.0 - fx) * (1.0 - fy)),
      (1, 0, fx * (1.0 - fy)),
      (0, 1, (1.0 - fx) * fy),
      (1, 1, fx * fy),
  )
  for c, (dx, dy, wgt) in enumerate(corners):
    xi = x0 + dx
    yi = y0 + dy
    valid = ((xi >= 0) & (xi <= wl - 1) & (yi >= 0) & (yi <= hl - 1))
    xc = jnp.clip(xi, 0, wl - 1)
    yc = jnp.clip(yi, 0, hl - 1)
    iw_ref[c] = (st + yc * wl + xc) * H_ + hh + base
    wu = lax.bitcast_convert_type(
        (aw * wgt * valid.astype(jnp.float32)).astype(jnp.bfloat16),
        jnp.uint16).astype(jnp.uint32)
    iw_ref[4 + c] = lax.bitcast_convert_type((wu << 16) | wu, jnp.int32)


def _sampling(q2, rpx, rpy, wox, woy, wat, box, boy, bat):
  return pl.pallas_call(
      _sampling_body,
      grid=(NBLK_,),
      in_specs=[
          pl.BlockSpec((TQ_, D_), lambda g: (g, 0)),
          pl.BlockSpec((TQ_, 128), lambda g: (g, 0)),
          pl.BlockSpec((TQ_, 128), lambda g: (g, 0)),
          pl.BlockSpec((D_, 128), lambda g: (0, 0)),
          pl.BlockSpec((D_, 128), lambda g: (0, 0)),
          pl.BlockSpec((D_, 128), lambda g: (0, 0)),
          pl.BlockSpec((1, 128), lambda g: (0, 0)),
          pl.BlockSpec((1, 128), lambda g: (0, 0)),
          pl.BlockSpec((1, 128), lambda g: (0, 0)),
          pl.BlockSpec((128, 128), lambda g: (0, 0)),
          pl.BlockSpec((4, 128), lambda g: (0, 0)),
      ],
      out_specs=pl.BlockSpec((8, TQ_, 128), lambda g: (0, g, 0)),
      out_shape=jax.ShapeDtypeStruct((8, NG_, 128), jnp.int32),
  )(q2, rpx, rpy, wox, woy, wat, box, boy, bat, jnp.asarray(_BGRP),
    jnp.asarray(np.stack([_W_I, _H_I, _START_I, _HEAD_I])))


_NS = 8


def _sc_gather(table, iw):
  mesh = plsc.VectorSubcoreMesh(core_axis_name="c", subcore_axis_name="s")
  nit = GPW_ // _NS

  @functools.partial(
      pl.kernel,
      out_type=jax.ShapeDtypeStruct((NG_, D_), jnp.float32),
      mesh=mesh,
      scratch_types=[
          [pltpu.VMEM((8, 128), jnp.int32)] * _NS,
          [[pltpu.VMEM((128, HD_), jnp.bfloat16)] * 4] * _NS,
          [pltpu.VMEM((D_,), jnp.float32)] * _NS,
          [pltpu.SemaphoreType.DMA] * _NS,
          [pltpu.SemaphoreType.DMA] * _NS,
          [pltpu.SemaphoreType.DMA] * _NS,
      ],
      compiler_params=pltpu.CompilerParams(use_tc_tiling_on_sc=False,
                                           needs_layout_passes=False),
  )
  def k(table_hbm, iw_hbm, out_hbm, iws, rows, os, sg, st, so):
    wid = lax.axis_index("s") * 2 + lax.axis_index("c")
    base = wid * GPW_

    def stage(g, s):
      pltpu.async_copy(iw_hbm.at[:, g], iws[s], st[s])

    def wait_stage(g, s):
      pltpu.make_async_copy(iw_hbm.at[:, g], iws[s], st[s]).wait()

    def gathers(s):
      for c in range(4):
        pltpu.async_copy(table_hbm.at[iws[s].at[c]], rows[s][c], sg[s])

    def wait_gathers(s):
      for c in range(4):
        pltpu.make_async_copy(table_hbm.at[iws[s].at[c]], rows[s][c], sg[s]).wait()

    def compute(s):
      w_v = iws[s]
      out_v = os[s]
      for h in range(H_):
        acc = []
        for c in range(4):
          wv = w_v[4 + c, pl.ds(h * 16, 16)]
          rr = rows[s][c]
          pc = [jnp.zeros((32,), jnp.bfloat16), jnp.zeros((32,), jnp.bfloat16)]
          for j in range(16):
            wj = lax.gather(
                wv, jnp.full((16, 1), j, jnp.int32),
                lax.GatherDimensionNumbers(offset_dims=(), collapsed_slice_dims=(0,),
                                           start_index_map=(0,)),
                (1,), mode=lax.GatherScatterMode.PROMISE_IN_BOUNDS)
            wj32 = plsc.bitcast(wj, jnp.bfloat16)
            pc[j % 2] = pc[j % 2] + wj32 * rr[h * 16 + j, :]
          acc.append(pc[0] + pc[1])
        e0, o0 = plsc.unpack(acc[0], format=plsc.PackFormat.INTERLEAVED)
        e1, o1 = plsc.unpack(acc[1], format=plsc.PackFormat.INTERLEAVED)
        e2, o2 = plsc.unpack(acc[2], format=plsc.PackFormat.INTERLEAVED)
        e3, o3 = plsc.unpack(acc[3], format=plsc.PackFormat.INTERLEAVED)
        out_v[pl.ds(h * HD_, 16)] = (e0 + e1) + (e2 + e3)
        out_v[pl.ds(h * HD_ + 16, 16)] = (o0 + o1) + (o2 + o3)

    for s in range(_NS):
      stage(base + s, s)
    wait_stage(base, 0)
    gathers(0)

    def body(i, carry):
      g0 = base + _NS * i
      for s in range(_NS):
        g = g0 + s
        sn = (s + 1) % _NS
        if s < _NS - 1:
          wait_stage(g + 1, sn)
          gathers(sn)
        else:
          @pl.when(i < nit - 1)
          def _():
            wait_stage(g + 1, sn)
            gathers(sn)
        wait_gathers(s)

        @pl.when(i > 0)
        def _():
          pltpu.make_async_copy(os[s], out_hbm.at[g - _NS], so[s]).wait()
        compute(s)
        pltpu.async_copy(os[s], out_hbm.at[g], so[s])

        @pl.when(i < nit - 1)
        def _():
          stage(g + _NS, s)
      return carry

    lax.fori_loop(0, nit, body, 0)
    for s in range(_NS):
      pltpu.make_async_copy(os[s], out_hbm.at[base + GPW_ - _NS + s], so[s]).wait()

  return k(table, iw)


def kernel(query, value, reference_points, spatial_shapes, level_start_index,
           W_value, b_value, W_offsets, b_offsets, W_attn, b_attn, W_out, b_out):
  q2 = query.reshape(NG_, D_)
  v2 = value.reshape(BS_ * NV_, D_)

  cperm = (np.arange(D_).reshape(H_, 2, 16).transpose(0, 2, 1).reshape(-1))
  table = _proj(v2, W_value[:, cperm], b_value[cperm].reshape(1, D_),
                out_dtype=jnp.bfloat16).reshape(BS_ * NV_ * H_, HD_)

  wox = W_offsets[:, 0::2]
  woy = W_offsets[:, 1::2]
  box = b_offsets[0::2].reshape(1, 128)
  boy = b_offsets[1::2].reshape(1, 128)
  bat = b_attn.reshape(1, 128)
  lane_l = jnp.asarray(_l_of.astype(np.int32))
  wl_f = jnp.asarray(_W_I.astype(np.float32))[None, None, :]
  hl_f = jnp.asarray(_H_I.astype(np.float32))[None, None, :]
  rpx = (jnp.take(reference_points[..., 0], lane_l, axis=2) * wl_f - 0.5).reshape(NG_, 128)
  rpy = (jnp.take(reference_points[..., 1], lane_l, axis=2) * hl_f - 0.5).reshape(NG_, 128)
  iw = _sampling(q2, rpx, rpy, wox, woy, W_attn, box, boy, bat)

  sampled = _sc_gather(table, iw)

  out = _out_proj(sampled, W_out, b_out.reshape(1, D_), q2)
  return out.reshape(BS_, NQ_, D_)


# speedup vs baseline: 1.2244x; 1.1001x over previous
import functools

import jax
import jax.numpy as jnp
import numpy as np
from jax import lax
from jax.experimental import pallas as pl
from jax.experimental.pallas import tpu as pltpu
from jax.experimental.pallas import tpu_sc as plsc

BS_, NQ_, D_ = 4, 5440, 256
H_, L_, P_ = 8, 4, 4
HD_ = D_ // H_
SH_ = np.array([[64, 64], [32, 32], [16, 16], [8, 8]], dtype=np.int64)
NV_ = int((SH_[:, 0] * SH_[:, 1]).sum())
STARTS_ = np.concatenate([[0], np.cumsum(SH_[:, 0] * SH_[:, 1])[:-1]]).astype(np.int64)
NG_ = BS_ * NQ_
TQ_ = 1360
NBLK_ = NG_ // TQ_
NW_ = 32
GPW_ = NG_ // NW_

_lane = np.arange(H_ * L_ * P_)
_l_of = (_lane // P_) % L_
_W_I = SH_[_l_of, 1].astype(np.int32)
_H_I = SH_[_l_of, 0].astype(np.int32)
_START_I = STARTS_[_l_of].astype(np.int32)
_HEAD_I = (_lane // (L_ * P_)).astype(np.int32)
_BGRP = (np.arange(128)[:, None] // (L_ * P_) == np.arange(128)[None, :] // (L_ * P_)).astype(np.float32)


def _matmul_body(x_ref, w_ref, b_ref, o_ref):
  r = jnp.dot(x_ref[...], w_ref[...], preferred_element_type=jnp.float32) + b_ref[0]
  o_ref[...] = r.astype(o_ref.dtype)


def _proj(x, w, b, out_dtype=jnp.float32):
  n = x.shape[0]
  return pl.pallas_call(
      _matmul_body,
      grid=(n // TQ_,),
      in_specs=[
          pl.BlockSpec((TQ_, x.shape[1]), lambda g: (g, 0)),
          pl.BlockSpec(w.shape, lambda g: (0, 0)),
          pl.BlockSpec((1, b.shape[1]), lambda g: (0, 0)),
      ],
      out_specs=pl.BlockSpec((TQ_, w.shape[1]), lambda g: (g, 0)),
      out_shape=jax.ShapeDtypeStruct((n, w.shape[1]), out_dtype),
  )(x, w, b)


def _residual_body(x_ref, w_ref, b_ref, q_ref, o_ref):
  o_ref[...] = (jnp.dot(x_ref[...], w_ref[...], preferred_element_type=jnp.float32)
                + b_ref[0] + q_ref[...])


def _out_proj(x, w, b, q):
  n = x.shape[0]
  return pl.pallas_call(
      _residual_body,
      grid=(n // TQ_,),
      in_specs=[
          pl.BlockSpec((TQ_, D_), lambda g: (g, 0)),
          pl.BlockSpec((D_, D_), lambda g: (0, 0)),
          pl.BlockSpec((1, D_), lambda g: (0, 0)),
          pl.BlockSpec((TQ_, D_), lambda g: (g, 0)),
      ],
      out_specs=pl.BlockSpec((TQ_, D_), lambda g: (g, 0)),
      out_shape=jax.ShapeDtypeStruct((n, D_), jnp.float32),
  )(x, w, b, q)


def _sampling_body(q_ref, rpx_ref, rpy_ref, wox_ref, woy_ref, wat_ref,
                   box_ref, boy_ref, bat_ref, bgrp_ref, lc_ref, iw_ref):
  q = q_ref[...]
  offx = jnp.dot(q, wox_ref[...], preferred_element_type=jnp.float32) + box_ref[0]
  offy = jnp.dot(q, woy_ref[...], preferred_element_type=jnp.float32) + boy_ref[0]
  a = jnp.dot(q, wat_ref[...], preferred_element_type=jnp.float32) + bat_ref[0]
  m = jnp.max(a, axis=-1, keepdims=True)
  e = jnp.exp(a - m)
  s = jnp.dot(e, bgrp_ref[...], preferred_element_type=jnp.float32)
  aw = e / s

  x = rpx_ref[...] + offx
  y = rpy_ref[...] + offy
  x0f = jnp.floor(x)
  y0f = jnp.floor(y)
  fx = x - x0f
  fy = y - y0f
  x0 = x0f.astype(jnp.int32)
  y0 = y0f.astype(jnp.int32)

  wl = lc_ref[0:1, :]
  hl = lc_ref[1:2, :]
  st = lc_ref[2:3, :]
  hh = lc_ref[3:4, :]
  b = pl.program_id(0) // (NQ_ // TQ_)
  base = (b * (NV_ * H_)).astype(jnp.int32)

  corners = (
      (0, 0, (1.0 - fx) * (1.0 - fy)),
      (1, 0, fx * (1.0 - fy)),
      (0, 1, (1.0 - fx) * fy),
      (1, 1, fx * fy),
  )
  for c, (dx, dy, wgt) in enumerate(corners):
    xi = x0 + dx
    yi = y0 + dy
    valid = ((xi >= 0) & (xi <= wl - 1) & (yi >= 0) & (yi <= hl - 1))
    xc = jnp.clip(xi, 0, wl - 1)
    yc = jnp.clip(yi, 0, hl - 1)
    iw_ref[c] = (st + yc * wl + xc) * H_ + hh + base
    wu = lax.bitcast_convert_type(
        (aw * wgt * valid.astype(jnp.float32)).astype(jnp.bfloat16),
        jnp.uint16).astype(jnp.uint32)
    iw_ref[4 + c] = lax.bitcast_convert_type((wu << 16) | wu, jnp.int32)


def _sampling(q2, rpx, rpy, wox, woy, wat, box, boy, bat):
  return pl.pallas_call(
      _sampling_body,
      grid=(NBLK_,),
      in_specs=[
          pl.BlockSpec((TQ_, D_), lambda g: (g, 0)),
          pl.BlockSpec((TQ_, 128), lambda g: (g, 0)),
          pl.BlockSpec((TQ_, 128), lambda g: (g, 0)),
          pl.BlockSpec((D_, 128), lambda g: (0, 0)),
          pl.BlockSpec((D_, 128), lambda g: (0, 0)),
          pl.BlockSpec((D_, 128), lambda g: (0, 0)),
          pl.BlockSpec((1, 128), lambda g: (0, 0)),
          pl.BlockSpec((1, 128), lambda g: (0, 0)),
          pl.BlockSpec((1, 128), lambda g: (0, 0)),
          pl.BlockSpec((128, 128), lambda g: (0, 0)),
          pl.BlockSpec((4, 128), lambda g: (0, 0)),
      ],
      out_specs=pl.BlockSpec((8, TQ_, 128), lambda g: (0, g, 0)),
      out_shape=jax.ShapeDtypeStruct((8, NG_, 128), jnp.int32),
  )(q2, rpx, rpy, wox, woy, wat, box, boy, bat, jnp.asarray(_BGRP),
    jnp.asarray(np.stack([_W_I, _H_I, _START_I, _HEAD_I])))


_NS = 4
_GU = 2


def _sc_gather(table, iw):
  mesh = plsc.VectorSubcoreMesh(core_axis_name="c", subcore_axis_name="s")
  npair = GPW_ // _GU
  nit = npair // _NS

  @functools.partial(
      pl.kernel,
      out_type=jax.ShapeDtypeStruct((NG_, D_), jnp.float32),
      mesh=mesh,
      scratch_types=[
          [pltpu.VMEM((8, _GU, 128), jnp.int32)] * _NS,
          [[[pltpu.VMEM((128, HD_), jnp.bfloat16)] * 4] * _GU] * _NS,
          [pltpu.VMEM((_GU, D_), jnp.float32)] * _NS,
          [pltpu.SemaphoreType.DMA] * _NS,
          [pltpu.SemaphoreType.DMA] * _NS,
          [pltpu.SemaphoreType.DMA] * _NS,
      ],
      compiler_params=pltpu.CompilerParams(use_tc_tiling_on_sc=False,
                                           needs_layout_passes=False),
  )
  def k(table_hbm, iw_hbm, out_hbm, iws, rows, os, sg, st, so):
    wid = lax.axis_index("s") * 2 + lax.axis_index("c")
    base = wid * npair

    def stage(p, s):
      pltpu.async_copy(iw_hbm.at[:, pl.ds(p * _GU, _GU)], iws[s], st[s])

    def wait_stage(p, s):
      pltpu.make_async_copy(iw_hbm.at[:, pl.ds(p * _GU, _GU)], iws[s], st[s]).wait()

    def gathers(s):
      for u in range(_GU):
        for c in range(4):
          pltpu.async_copy(table_hbm.at[iws[s].at[c, u]], rows[s][u][c], sg[s])

    def wait_gathers(s):
      for u in range(_GU):
        for c in range(4):
          pltpu.make_async_copy(table_hbm.at[iws[s].at[c, u]], rows[s][u][c],
                                sg[s]).wait()

    def compute(s):
      w_v = iws[s]
      out_v = os[s]
      for u in range(_GU):
        for h in range(H_):
          acc = []
          for c in range(4):
            wv = w_v[4 + c, u, pl.ds(h * 16, 16)]
            rr = rows[s][u][c]
            pc = [jnp.zeros((32,), jnp.bfloat16), jnp.zeros((32,), jnp.bfloat16)]
            for j in range(16):
              wj = lax.gather(
                  wv, jnp.full((16, 1), j, jnp.int32),
                  lax.GatherDimensionNumbers(offset_dims=(), collapsed_slice_dims=(0,),
                                             start_index_map=(0,)),
                  (1,), mode=lax.GatherScatterMode.PROMISE_IN_BOUNDS)
              wj32 = plsc.bitcast(wj, jnp.bfloat16)
              pc[j % 2] = pc[j % 2] + wj32 * rr[h * 16 + j, :]
            acc.append(pc[0] + pc[1])
          e0, o0 = plsc.unpack(acc[0], format=plsc.PackFormat.INTERLEAVED)
          e1, o1 = plsc.unpack(acc[1], format=plsc.PackFormat.INTERLEAVED)
          e2, o2 = plsc.unpack(acc[2], format=plsc.PackFormat.INTERLEAVED)
          e3, o3 = plsc.unpack(acc[3], format=plsc.PackFormat.INTERLEAVED)
          out_v[u, pl.ds(h * HD_, 16)] = (e0 + e1) + (e2 + e3)
          out_v[u, pl.ds(h * HD_ + 16, 16)] = (o0 + o1) + (o2 + o3)

    for s in range(_NS):
      stage(base + s, s)
    wait_stage(base, 0)
    gathers(0)

    def body(i, carry):
      p0 = base + _NS * i
      for s in range(_NS):
        p = p0 + s
        sn = (s + 1) % _NS
        if s < _NS - 1:
          wait_stage(p + 1, sn)
          gathers(sn)
        else:
          @pl.when(i < nit - 1)
          def _():
            wait_stage(p + 1, sn)
            gathers(sn)
        wait_gathers(s)

        @pl.when(i > 0)
        def _():
          pltpu.make_async_copy(os[s], out_hbm.at[pl.ds((p - _NS) * _GU, _GU)],
                                so[s]).wait()
        compute(s)
        pltpu.async_copy(os[s], out_hbm.at[pl.ds(p * _GU, _GU)], so[s])

        @pl.when(i < nit - 1)
        def _():
          stage(p + _NS, s)
      return carry

    lax.fori_loop(0, nit, body, 0)
    for s in range(_NS):
      pltpu.make_async_copy(
          os[s], out_hbm.at[pl.ds((base + npair - _NS + s) * _GU, _GU)],
          so[s]).wait()

  return k(table, iw)


def kernel(query, value, reference_points, spatial_shapes, level_start_index,
           W_value, b_value, W_offsets, b_offsets, W_attn, b_attn, W_out, b_out):
  q2 = query.reshape(NG_, D_)
  v2 = value.reshape(BS_ * NV_, D_)

  cperm = (np.arange(D_).reshape(H_, 2, 16).transpose(0, 2, 1).reshape(-1))
  table = _proj(v2, W_value[:, cperm], b_value[cperm].reshape(1, D_),
                out_dtype=jnp.bfloat16).reshape(BS_ * NV_ * H_, HD_)

  wox = W_offsets[:, 0::2]
  woy = W_offsets[:, 1::2]
  box = b_offsets[0::2].reshape(1, 128)
  boy = b_offsets[1::2].reshape(1, 128)
  bat = b_attn.reshape(1, 128)
  lane_l = jnp.asarray(_l_of.astype(np.int32))
  wl_f = jnp.asarray(_W_I.astype(np.float32))[None, None, :]
  hl_f = jnp.asarray(_H_I.astype(np.float32))[None, None, :]
  rpx = (jnp.take(reference_points[..., 0], lane_l, axis=2) * wl_f - 0.5).reshape(NG_, 128)
  rpy = (jnp.take(reference_points[..., 1], lane_l, axis=2) * hl_f - 0.5).reshape(NG_, 128)
  iw = _sampling(q2, rpx, rpy, wox, woy, W_attn, box, boy, bat)

  sampled = _sc_gather(table, iw)

  out = _out_proj(sampled, W_out, b_out.reshape(1, D_), q2)
  return out.reshape(BS_, NQ_, D_)
